# Initial kernel scaffold; baseline (speedup 1.0000x reference)
import jax
import jax.numpy as jnp
from jax.experimental import pallas as pl


def kernel(scores, pairs, unpair0, unpair1):
    raise NotImplementedError("write your pallas kernel here")


# speedup vs baseline: 3.1696x; 3.1696x over previous
import functools

import jax
import jax.numpy as jnp
from jax import lax
from jax.experimental import pallas as pl
from jax.experimental.pallas import tpu as pltpu
from jax.experimental.pallas import tpu_sc as plsc

_NC = 2
_NS = 16
_L = 16
_NW = _NC * _NS


def _make_sc_gather_loss(B, N, M, P, U, interpret=False):
    total = B * (P + 2 * U)
    assert total % _NW == 0
    n_per_w = total // _NW
    assert n_per_w % (8 * _L) == 0
    n_chunks = n_per_w // 128
    pair_entries = B * P
    u0_entries = B * U
    assert pair_entries % n_per_w == 0 and u0_entries % n_per_w == 0
    pair_workers = pair_entries // n_per_w
    u_workers = u0_entries // n_per_w
    batches_per_pair_worker = pair_workers // B

    w_pair = -1.0 / (P * B)
    w_unpair = -1.0 / (U * B)

    mesh = plsc.VectorSubcoreMesh(
        core_axis_name="c", subcore_axis_name="s", num_cores=_NC,
        num_subcores=_NS)

    @functools.partial(
        pl.kernel,
        out_type=jax.ShapeDtypeStruct((_NW, _L), jnp.float32),
        mesh=mesh,
        scratch_types=[
            pltpu.VMEM((n_per_w,), jnp.int32),
            pltpu.VMEM((n_per_w,), jnp.int32),
            pltpu.VMEM((n_chunks, 128), jnp.int32),
            pltpu.VMEM((n_chunks, 128), jnp.float32),
            pltpu.VMEM((_L,), jnp.float32),
            pltpu.SemaphoreType.DMA,
        ],
        interpret=interpret,
    )
    def sc_loss(rows_hbm, cols_hbm, scores_hbm, out_hbm,
                rows_v, cols_v, idx_v, vals_v, tmp_v, sem):
        cid = lax.axis_index("c")
        sid = lax.axis_index("s")
        wid = sid * _NC + cid

        batch = jnp.where(
            wid < pair_workers,
            wid // batches_per_pair_worker,
            jnp.where(wid < pair_workers + u_workers,
                      wid - pair_workers,
                      wid - (pair_workers + u_workers)))
        base = batch * (N * M)
        weight = jnp.where(wid < pair_workers,
                           jnp.float32(w_pair), jnp.float32(w_unpair))

        off = wid * n_per_w
        pltpu.sync_copy(rows_hbm.at[pl.ds(off, n_per_w)], rows_v)
        pltpu.sync_copy(cols_hbm.at[pl.ds(off, n_per_w)], cols_v)

        for j in range(n_per_w // _L):
            r = rows_v[pl.ds(j * _L, _L)]
            c = cols_v[pl.ds(j * _L, _L)]
            idx_v[j // 8, pl.ds((j % 8) * _L, _L)] = base + r * M + c

        copies = [
            pltpu.async_copy(scores_hbm.at[idx_v.at[j]], vals_v.at[j], sem)
            for j in range(n_chunks)
        ]
        for cp in copies:
            cp.wait()

        acc = jnp.zeros((_L,), jnp.float32)
        for j in range(n_chunks):
            for k in range(128 // _L):
                acc = acc + vals_v[j, pl.ds(k * _L, _L)]
        tmp_v[...] = acc * weight

        pltpu.sync_copy(tmp_v, out_hbm.at[wid])

    return sc_loss


def kernel(scores, pairs, unpair0, unpair1):
    B, N, M = scores.shape
    P = pairs.shape[1]
    U = unpair0.shape[1]

    rows = jnp.concatenate([
        pairs[..., 0].reshape(-1),
        unpair0.reshape(-1),
        jnp.full((B * U,), N - 1, jnp.int32),
    ])
    cols = jnp.concatenate([
        pairs[..., 1].reshape(-1),
        jnp.full((B * U,), M - 1, jnp.int32),
        unpair1.reshape(-1),
    ])

    sc_loss = _make_sc_gather_loss(B, N, M, P, U)
    out = sc_loss(rows, cols, scores.reshape(-1))
    return jnp.sum(out)


# speedup vs baseline: 5.3792x; 1.6971x over previous
import functools

import jax
import jax.numpy as jnp
from jax import lax
from jax.experimental import pallas as pl
from jax.experimental.pallas import tpu as pltpu
from jax.experimental.pallas import tpu_sc as plsc

_NC = 2
_NS = 16
_L = 16
_NW = _NC * _NS


def _make_sc_gather_loss(B, N, M, P, U):
    n_elems = B * (P + U)
    assert n_elems % _NW == 0
    n_per_w = n_elems // _NW
    assert n_per_w % _L == 0
    chunk = 16
    assert n_per_w % chunk == 0
    n_chunks = n_per_w // chunk
    n_u1 = B * U
    assert n_u1 % _NW == 0
    u1_per_w = n_u1 // _NW
    assert _NW % B == 0
    wpb = _NW // B

    mesh = plsc.VectorSubcoreMesh(
        core_axis_name="c", subcore_axis_name="s", num_cores=_NC,
        num_subcores=_NS)

    @functools.partial(
        pl.kernel,
        out_type=jax.ShapeDtypeStruct((_NW, _L), jnp.float32),
        mesh=mesh,
        scratch_types=[
            pltpu.VMEM((n_per_w,), jnp.int32),
            pltpu.VMEM((n_per_w,), jnp.int32),
            pltpu.VMEM((n_per_w,), jnp.float32),
            pltpu.VMEM((n_chunks, chunk), jnp.int32),
            pltpu.VMEM((2, chunk, M), jnp.float32),
            pltpu.VMEM((1, M), jnp.float32),
            pltpu.VMEM((u1_per_w,), jnp.int32),
            pltpu.VMEM((n_per_w,), jnp.float32),
            pltpu.VMEM((_L,), jnp.float32),
            pltpu.SemaphoreType.DMA,
            pltpu.SemaphoreType.DMA,
        ],
        compiler_params=pltpu.CompilerParams(needs_layout_passes=False),
    )
    def sc_loss(rows_hbm, cols_hbm, wgt_hbm, u1_hbm, scores_hbm, out_hbm,
                rows_v, cols_v, wgt_v, ridx_v, buf_v, u1row_v, u1c_v, sel_v,
                tmp_v, sem0, sem1):
        cid = lax.axis_index("c")
        sid = lax.axis_index("s")
        wid = sid * _NC + cid
        s2d = scores_hbm.reshape(B * N, M)
        sems = [sem0, sem1]

        off = wid * n_per_w
        pltpu.sync_copy(rows_hbm.at[pl.ds(off, n_per_w)], rows_v)
        pltpu.sync_copy(cols_hbm.at[pl.ds(off, n_per_w)], cols_v)
        pltpu.sync_copy(wgt_hbm.at[pl.ds(off, n_per_w)], wgt_v)
        for j in range(n_per_w // _L):
            r = rows_v[pl.ds(j * _L, _L)]
            ridx_v[j // (chunk // _L), pl.ds((j % (chunk // _L)) * _L, _L)] = r

        def start(k):
            return pltpu.async_copy(
                s2d.at[ridx_v.at[k]], buf_v.at[k % 2], sems[k % 2])

        cps = {0: start(0)}
        for k in range(n_chunks):
            if k + 1 < n_chunks:
                cps[k + 1] = start(k + 1)
            cps.pop(k).wait()
            for g in range(chunk // _L):
                rowloc = jax.lax.iota(jnp.int32, _L) + g * _L
                lanes = cols_v[pl.ds(k * chunk + g * _L, _L)]
                w = wgt_v[pl.ds(k * chunk + g * _L, _L)]
                sel_v[pl.ds(k * chunk + g * _L, _L)] = (
                    w * plsc.load_gather(buf_v.at[k % 2], [rowloc, lanes]))
        acc = jnp.zeros((_L,), jnp.float32)
        for j in range(n_per_w // _L):
            acc = acc + sel_v[pl.ds(j * _L, _L)]

        batch = wid // wpb
        q = wid % wpb
        pltpu.sync_copy(s2d.at[pl.ds(batch * N + N - 1, 1), pl.ds(0, M)],
                        u1row_v)
        pltpu.sync_copy(
            u1_hbm.at[pl.ds(batch * U + q * u1_per_w, u1_per_w)], u1c_v)
        uacc = jnp.zeros((_L,), jnp.float32)
        zero16 = jnp.zeros((_L,), jnp.int32)
        for j in range(u1_per_w // _L):
            c = u1c_v[pl.ds(j * _L, _L)]
            uacc = uacc + plsc.load_gather(u1row_v, [zero16, c])

        w_unpair = jnp.float32(-1.0 / (U * B))
        tmp_v[...] = acc + w_unpair * uacc
        pltpu.sync_copy(tmp_v, out_hbm.at[wid])

    return sc_loss


def kernel(scores, pairs, unpair0, unpair1):
    B, N, M = scores.shape
    P = pairs.shape[1]
    U = unpair0.shape[1]

    gbase = (jnp.arange(B, dtype=jnp.int32) * N)[:, None]
    rows = jnp.concatenate([
        (pairs[..., 0] + gbase).reshape(-1),
        (unpair0 + gbase).reshape(-1),
    ])
    cols = jnp.concatenate([
        pairs[..., 1].reshape(-1),
        jnp.full((B * U,), M - 1, jnp.int32),
    ])
    w_pair = -1.0 / (P * B)
    w_unpair = -1.0 / (U * B)
    wgt = jnp.concatenate([
        jnp.full((B * P,), w_pair, jnp.float32),
        jnp.full((B * U,), w_unpair, jnp.float32),
    ])

    sc_loss = _make_sc_gather_loss(B, N, M, P, U)
    out = sc_loss(rows, cols, wgt, unpair1.reshape(-1), scores)
    return jnp.sum(out)
